# baseline (speedup 1.0000x reference)
import jax
import jax.numpy as jnp
from jax import lax
from jax.experimental import pallas as pl
from jax.experimental.pallas import tpu as pltpu
from jax.experimental.pallas import tpu_sc as plsc

KNN = 20
RBLK = 512
NEG = -1e30

_NW = 32
_CHUNK = 32


def _tc_body(xf_ref, xb_ref, w_ref, y1t_ref, y2t_ref, idx_ref, nnt_ref,
             dist_ref):
    b = pl.program_id(0)
    r = pl.program_id(1)
    xfull = xf_ref[0]
    xr = xb_ref[0]
    n = xfull.shape[1]
    w1 = w_ref[:, :64]
    w2m1 = w_ref[:, 64:] - w1

    @pl.when(r == 0)
    def _():
        xsq = xfull * xfull
        ones = jnp.ones((xfull.shape[0], 8), jnp.float32)
        nnt_ref[...] = lax.dot_general(xsq, ones, (((0,), (0,)), ((), ())),
                                       precision=lax.Precision.HIGHEST,
                                       preferred_element_type=jnp.float32)

    w1cat = jnp.concatenate([w1, w1], axis=0)
    y1t_ref[...] = lax.dot_general(xr, w1cat, (((0,), (1,)), ((), ())),
                                   preferred_element_type=jnp.float32)
    y2t_ref[...] = lax.dot_general(xr, w2m1, (((0,), (1,)), ((), ())),
                                   preferred_element_type=jnp.float32)

    rows = lax.broadcasted_iota(jnp.int32, (n, RBLK), 0)
    cols = lax.broadcasted_iota(jnp.int32, (n, RBLK), 1)
    nblk = n // RBLK
    base = b * n

    def stage(dst, xcols, blk):
        g2 = lax.dot_general(xfull, xcols + xcols, (((0,), (0,)), ((), ())),
                             preferred_element_type=jnp.float32)
        dist_ref[dst] = jnp.where(rows == cols + blk * RBLK, NEG,
                                  g2 - nnt_ref[:, 0:1])

    @pl.when(r == 0)
    def _():
        stage(0, xr, r)

    @pl.when(r + 1 < nblk)
    def _():
        xnext = xf_ref[0, :, pl.ds(pl.multiple_of((r + 1) * RBLK, RBLK), RBLK)]
        stage((r + 1) % 2, xnext, r + 1)

    r2 = r % 2
    idx_ref[0:1, :] = lax.broadcasted_iota(jnp.int32, (1, RBLK), 1) + (
        base + r * RBLK)
    jprev = None
    for s in range(1, KNN):
        d = dist_ref[r2]
        if jprev is not None:
            d = jnp.where(rows == jprev, NEG, d)
            dist_ref[r2] = d
        jmin = jnp.argmax(d, axis=0, keepdims=True).astype(jnp.int32)
        idx_ref[s:s + 1, :] = jmin + base
        jprev = jmin


def _sc_body(y1t_hbm, idx_hbm, y2t_hbm, out_hbm, idx_v, rows_v, y2_v, out_v,
             sem):
    wid = lax.axis_index("s") * 2 + lax.axis_index("c")
    pts = y1t_hbm.shape[0] // _NW
    tile_base = wid * pts
    pltpu.sync_copy(idx_hbm.at[pl.ds(0, 24), pl.ds(tile_base, pts)], idx_v)

    def chunk(ch, _):
        base = tile_base + ch * _CHUNK
        off = ch * _CHUNK
        copies = [
            pltpu.async_copy(y1t_hbm.at[idx_v.at[s, pl.ds(off, _CHUNK)]],
                             rows_v.at[s], sem)
            for s in range(KNN)
        ]
        pltpu.sync_copy(y2t_hbm.at[pl.ds(base, _CHUNK)], y2_v)
        for c in copies:
            c.wait()

        def point(p, _):
            for c in range(4):
                sl = pl.ds(c * 16, 16)
                acc = rows_v[0, p, sl]
                for s in range(1, KNN):
                    acc = jnp.maximum(acc, rows_v[s, p, sl])
                z = acc + y2_v[p, sl]
                out_v[p, sl] = jnp.where(z >= 0, z, 0.01 * z)
            return 0

        lax.fori_loop(0, _CHUNK, point, 0)
        pltpu.sync_copy(out_v, out_hbm.at[pl.ds(base, _CHUNK)])
        return 0

    lax.fori_loop(0, pts // _CHUNK, chunk, 0)


def _half(xh, W):
    B, C, N = xh.shape
    O = W.shape[0]
    nblk = N // RBLK

    y1t, y2t, idxt = pl.pallas_call(
        _tc_body,
        grid=(B, nblk),
        in_specs=[
            pl.BlockSpec((1, C, N), lambda b, r: (b, 0, 0)),
            pl.BlockSpec((1, C, RBLK), lambda b, r: (b, 0, r)),
            pl.BlockSpec((O, 2 * C), lambda b, r: (0, 0)),
        ],
        out_specs=[
            pl.BlockSpec((RBLK, 2 * C), lambda b, r: (b * (N // RBLK) + r, 0)),
            pl.BlockSpec((RBLK, C), lambda b, r: (b * (N // RBLK) + r, 0)),
            pl.BlockSpec((24, RBLK), lambda b, r: (0, b * (N // RBLK) + r)),
        ],
        out_shape=[
            jax.ShapeDtypeStruct((B * N, 2 * C), jnp.float32),
            jax.ShapeDtypeStruct((B * N, C), jnp.float32),
            jax.ShapeDtypeStruct((24, B * N), jnp.int32),
        ],
        scratch_shapes=[
            pltpu.VMEM((N, 8), jnp.float32),
            pltpu.VMEM((2, N, RBLK), jnp.float32),
        ],
        compiler_params=pltpu.CompilerParams(
            dimension_semantics=("arbitrary", "arbitrary")),
    )(xh, xh, W)

    mesh = plsc.VectorSubcoreMesh(core_axis_name="c", subcore_axis_name="s")
    outt = pl.kernel(
        _sc_body,
        mesh=mesh,
        out_type=jax.ShapeDtypeStruct((B * N, O), jnp.float32),
        scratch_types=[
            pltpu.VMEM((24, B * N // _NW), jnp.int32),
            pltpu.VMEM((KNN, _CHUNK, 2 * O), jnp.float32),
            pltpu.VMEM((_CHUNK, O), jnp.float32),
            pltpu.VMEM((_CHUNK, O), jnp.float32),
            pltpu.SemaphoreType.DMA,
        ],
    )(y1t, idxt, y2t)

    return outt.reshape(B, N, O)


def kernel(x, W):
    B, C, N = x.shape
    h = B // 4
    outs = [_half(x[i * h:(i + 1) * h], W) for i in range(4)]
    return jnp.concatenate(outs, axis=0).transpose(0, 2, 1)


# speedup vs baseline: 1.0054x; 1.0054x over previous
import jax
import jax.numpy as jnp
from jax import lax
from jax.experimental import pallas as pl
from jax.experimental.pallas import tpu as pltpu
from jax.experimental.pallas import tpu_sc as plsc

KNN = 20
RBLK = 256
NEG = -1e30

_NW = 32
_CHUNK = 32


def _tc_body(xf_ref, xb_ref, w_ref, y1t_ref, y2t_ref, idx_ref, nnt_ref,
             dist_ref):
    b = pl.program_id(0)
    r = pl.program_id(1)
    xfull = xf_ref[0]
    xr = xb_ref[0]
    n = xfull.shape[1]
    w1 = w_ref[:, :64]
    w2m1 = w_ref[:, 64:] - w1

    @pl.when(r == 0)
    def _():
        xsq = xfull * xfull
        ones = jnp.ones((xfull.shape[0], 8), jnp.float32)
        nnt_ref[...] = lax.dot_general(xsq, ones, (((0,), (0,)), ((), ())),
                                       precision=lax.Precision.HIGHEST,
                                       preferred_element_type=jnp.float32)

    w1cat = jnp.concatenate([w1, w1], axis=0)
    y1t_ref[...] = lax.dot_general(xr, w1cat, (((0,), (1,)), ((), ())),
                                   preferred_element_type=jnp.float32)
    y2t_ref[...] = lax.dot_general(xr, w2m1, (((0,), (1,)), ((), ())),
                                   preferred_element_type=jnp.float32)

    rows = lax.broadcasted_iota(jnp.int32, (n, RBLK), 0)
    cols = lax.broadcasted_iota(jnp.int32, (n, RBLK), 1)
    nblk = n // RBLK
    base = b * n

    def stage(dst, xcols, blk):
        g2 = lax.dot_general(xfull, xcols + xcols, (((0,), (0,)), ((), ())),
                             preferred_element_type=jnp.float32)
        dist_ref[dst] = jnp.where(rows == cols + blk * RBLK, NEG,
                                  g2 - nnt_ref[:, 0:1])

    @pl.when(r == 0)
    def _():
        stage(0, xr, r)

    @pl.when(r + 1 < nblk)
    def _():
        xnext = xf_ref[0, :, pl.ds(pl.multiple_of((r + 1) * RBLK, RBLK), RBLK)]
        stage((r + 1) % 2, xnext, r + 1)

    r2 = r % 2
    idx_ref[0:1, :] = lax.broadcasted_iota(jnp.int32, (1, RBLK), 1) + (
        base + r * RBLK)
    jprev = None
    for s in range(1, KNN):
        d = dist_ref[r2]
        if jprev is not None:
            d = jnp.where(rows == jprev, NEG, d)
            dist_ref[r2] = d
        jmin = jnp.argmax(d, axis=0, keepdims=True).astype(jnp.int32)
        idx_ref[s:s + 1, :] = jmin + base
        jprev = jmin


def _sc_body(y1t_hbm, idx_hbm, y2t_hbm, out_hbm, idx_v, rows_v, y2_v, out_v,
             sem):
    wid = lax.axis_index("s") * 2 + lax.axis_index("c")
    pts = y1t_hbm.shape[0] // _NW
    tile_base = wid * pts
    pltpu.sync_copy(idx_hbm.at[pl.ds(0, 24), pl.ds(tile_base, pts)], idx_v)

    def chunk(ch, _):
        base = tile_base + ch * _CHUNK
        off = ch * _CHUNK
        copies = [
            pltpu.async_copy(y1t_hbm.at[idx_v.at[s, pl.ds(off, _CHUNK)]],
                             rows_v.at[s], sem)
            for s in range(KNN)
        ]
        pltpu.sync_copy(y2t_hbm.at[pl.ds(base, _CHUNK)], y2_v)
        for c in copies:
            c.wait()

        def point(p, _):
            for c in range(4):
                sl = pl.ds(c * 16, 16)
                acc = rows_v[0, p, sl]
                for s in range(1, KNN):
                    acc = jnp.maximum(acc, rows_v[s, p, sl])
                z = acc + y2_v[p, sl]
                out_v[p, sl] = jnp.where(z >= 0, z, 0.01 * z)
            return 0

        lax.fori_loop(0, _CHUNK, point, 0)
        pltpu.sync_copy(out_v, out_hbm.at[pl.ds(base, _CHUNK)])
        return 0

    lax.fori_loop(0, pts // _CHUNK, chunk, 0)


def _half(xh, W):
    B, C, N = xh.shape
    O = W.shape[0]
    nblk = N // RBLK

    y1t, y2t, idxt = pl.pallas_call(
        _tc_body,
        grid=(B, nblk),
        in_specs=[
            pl.BlockSpec((1, C, N), lambda b, r: (b, 0, 0)),
            pl.BlockSpec((1, C, RBLK), lambda b, r: (b, 0, r)),
            pl.BlockSpec((O, 2 * C), lambda b, r: (0, 0)),
        ],
        out_specs=[
            pl.BlockSpec((RBLK, 2 * C), lambda b, r: (b * (N // RBLK) + r, 0)),
            pl.BlockSpec((RBLK, C), lambda b, r: (b * (N // RBLK) + r, 0)),
            pl.BlockSpec((24, RBLK), lambda b, r: (0, b * (N // RBLK) + r)),
        ],
        out_shape=[
            jax.ShapeDtypeStruct((B * N, 2 * C), jnp.float32),
            jax.ShapeDtypeStruct((B * N, C), jnp.float32),
            jax.ShapeDtypeStruct((24, B * N), jnp.int32),
        ],
        scratch_shapes=[
            pltpu.VMEM((N, 8), jnp.float32),
            pltpu.VMEM((2, N, RBLK), jnp.float32),
        ],
        compiler_params=pltpu.CompilerParams(
            dimension_semantics=("arbitrary", "arbitrary")),
    )(xh, xh, W)

    mesh = plsc.VectorSubcoreMesh(core_axis_name="c", subcore_axis_name="s")
    outt = pl.kernel(
        _sc_body,
        mesh=mesh,
        out_type=jax.ShapeDtypeStruct((B * N, O), jnp.float32),
        scratch_types=[
            pltpu.VMEM((24, B * N // _NW), jnp.int32),
            pltpu.VMEM((KNN, _CHUNK, 2 * O), jnp.float32),
            pltpu.VMEM((_CHUNK, O), jnp.float32),
            pltpu.VMEM((_CHUNK, O), jnp.float32),
            pltpu.SemaphoreType.DMA,
        ],
    )(y1t, idxt, y2t)

    return outt.reshape(B, N, O)


def kernel(x, W):
    B, C, N = x.shape
    h = B // 4
    outs = [_half(x[i * h:(i + 1) * h], W) for i in range(4)]
    return jnp.concatenate(outs, axis=0).transpose(0, 2, 1)


# speedup vs baseline: 1.0297x; 1.0242x over previous
import jax
import jax.numpy as jnp
from jax import lax
from jax.experimental import pallas as pl
from jax.experimental.pallas import tpu as pltpu
from jax.experimental.pallas import tpu_sc as plsc

KNN = 20
RBLK = 256
NEG = -1e30

_NW = 32
_CHUNK = 32


def _tc_body(xf_ref, xb_ref, w_ref, y1t_ref, y2t_ref, idx_ref, nnt_ref,
             dist_ref):
    b = pl.program_id(0)
    r = pl.program_id(1)
    xfull = xf_ref[0]
    xr = xb_ref[0]
    n = xfull.shape[1]
    w1 = w_ref[:, :64]
    w2m1 = w_ref[:, 64:] - w1

    @pl.when(r == 0)
    def _():
        xsq = xfull * xfull
        ones = jnp.ones((xfull.shape[0], 8), jnp.float32)
        nnt_ref[...] = lax.dot_general(xsq, ones, (((0,), (0,)), ((), ())),
                                       precision=lax.Precision.HIGHEST,
                                       preferred_element_type=jnp.float32)

    w1cat = jnp.concatenate([w1, w1], axis=0)
    y1t_ref[...] = lax.dot_general(xr, w1cat, (((0,), (1,)), ((), ())),
                                   preferred_element_type=jnp.float32)
    y2t_ref[...] = lax.dot_general(xr, w2m1, (((0,), (1,)), ((), ())),
                                   preferred_element_type=jnp.float32)

    rows = lax.broadcasted_iota(jnp.int32, (n, RBLK), 0)
    cols = lax.broadcasted_iota(jnp.int32, (n, RBLK), 1)
    nblk2 = n // (2 * RBLK)
    base = b * n

    def stage(dst, blk):
        xcols = xf_ref[0, :, pl.ds(pl.multiple_of(blk * RBLK, RBLK), RBLK)]
        g2 = lax.dot_general(xfull, xcols + xcols, (((0,), (0,)), ((), ())),
                             preferred_element_type=jnp.float32)
        dist_ref[dst] = jnp.where(rows == cols + blk * RBLK, NEG,
                                  g2 - nnt_ref[:, 0:1])

    @pl.when(r == 0)
    def _():
        stage(0, 0)
        stage(1, 1)

    @pl.when(r + 1 < nblk2)
    def _():
        stage((2 * r + 2) % 4, 2 * r + 2)
        stage((2 * r + 3) % 4, 2 * r + 3)

    i0 = 2 * (r % 2)
    i1 = i0 + 1
    idx_ref[0:1, :] = lax.broadcasted_iota(jnp.int32, (1, 2 * RBLK), 1) + (
        base + r * 2 * RBLK)
    jp0 = None
    jp1 = None
    for s in range(1, KNN):
        d0 = dist_ref[i0]
        d1 = dist_ref[i1]
        if jp0 is not None:
            d0 = jnp.where(rows == jp0, NEG, d0)
            dist_ref[i0] = d0
            d1 = jnp.where(rows == jp1, NEG, d1)
            dist_ref[i1] = d1
        j0 = jnp.argmax(d0, axis=0, keepdims=True).astype(jnp.int32)
        j1 = jnp.argmax(d1, axis=0, keepdims=True).astype(jnp.int32)
        idx_ref[s:s + 1, 0:RBLK] = j0 + base
        idx_ref[s:s + 1, RBLK:2 * RBLK] = j1 + base
        jp0 = j0
        jp1 = j1


def _sc_body(y1t_hbm, idx_hbm, y2t_hbm, out_hbm, idx_v, rows_v, y2_v, out_v,
             sem):
    wid = lax.axis_index("s") * 2 + lax.axis_index("c")
    pts = y1t_hbm.shape[0] // _NW
    tile_base = wid * pts
    pltpu.sync_copy(idx_hbm.at[pl.ds(0, 24), pl.ds(tile_base, pts)], idx_v)

    def chunk(ch, _):
        base = tile_base + ch * _CHUNK
        off = ch * _CHUNK
        copies = [
            pltpu.async_copy(y1t_hbm.at[idx_v.at[s, pl.ds(off, _CHUNK)]],
                             rows_v.at[s], sem)
            for s in range(KNN)
        ]
        pltpu.sync_copy(y2t_hbm.at[pl.ds(base, _CHUNK)], y2_v)
        for c in copies:
            c.wait()

        def point(p, _):
            for c in range(4):
                sl = pl.ds(c * 16, 16)
                acc = rows_v[0, p, sl]
                for s in range(1, KNN):
                    acc = jnp.maximum(acc, rows_v[s, p, sl])
                z = acc + y2_v[p, sl]
                out_v[p, sl] = jnp.where(z >= 0, z, 0.01 * z)
            return 0

        lax.fori_loop(0, _CHUNK, point, 0)
        pltpu.sync_copy(out_v, out_hbm.at[pl.ds(base, _CHUNK)])
        return 0

    lax.fori_loop(0, pts // _CHUNK, chunk, 0)


def _half(xh, W):
    B, C, N = xh.shape
    O = W.shape[0]
    nblk = N // RBLK

    dblk = 2 * RBLK
    y1t, y2t, idxt = pl.pallas_call(
        _tc_body,
        grid=(B, N // dblk),
        in_specs=[
            pl.BlockSpec((1, C, N), lambda b, r: (b, 0, 0)),
            pl.BlockSpec((1, C, dblk), lambda b, r: (b, 0, r)),
            pl.BlockSpec((O, 2 * C), lambda b, r: (0, 0)),
        ],
        out_specs=[
            pl.BlockSpec((dblk, 2 * C), lambda b, r: (b * (N // dblk) + r, 0)),
            pl.BlockSpec((dblk, C), lambda b, r: (b * (N // dblk) + r, 0)),
            pl.BlockSpec((24, dblk), lambda b, r: (0, b * (N // dblk) + r)),
        ],
        out_shape=[
            jax.ShapeDtypeStruct((B * N, 2 * C), jnp.float32),
            jax.ShapeDtypeStruct((B * N, C), jnp.float32),
            jax.ShapeDtypeStruct((24, B * N), jnp.int32),
        ],
        scratch_shapes=[
            pltpu.VMEM((N, 8), jnp.float32),
            pltpu.VMEM((4, N, RBLK), jnp.float32),
        ],
        compiler_params=pltpu.CompilerParams(
            dimension_semantics=("arbitrary", "arbitrary")),
    )(xh, xh, W)

    mesh = plsc.VectorSubcoreMesh(core_axis_name="c", subcore_axis_name="s")
    outt = pl.kernel(
        _sc_body,
        mesh=mesh,
        out_type=jax.ShapeDtypeStruct((B * N, O), jnp.float32),
        scratch_types=[
            pltpu.VMEM((24, B * N // _NW), jnp.int32),
            pltpu.VMEM((KNN, _CHUNK, 2 * O), jnp.float32),
            pltpu.VMEM((_CHUNK, O), jnp.float32),
            pltpu.VMEM((_CHUNK, O), jnp.float32),
            pltpu.SemaphoreType.DMA,
        ],
    )(y1t, idxt, y2t)

    return outt.reshape(B, N, O)


def kernel(x, W):
    B, C, N = x.shape
    h = B // 4
    outs = [_half(x[i * h:(i + 1) * h], W) for i in range(4)]
    return jnp.concatenate(outs, axis=0).transpose(0, 2, 1)


# speedup vs baseline: 1.0338x; 1.0039x over previous
import jax
import jax.numpy as jnp
from jax import lax
from jax.experimental import pallas as pl
from jax.experimental.pallas import tpu as pltpu
from jax.experimental.pallas import tpu_sc as plsc

KNN = 20
RBLK = 256
NEG = -1e30

_NW = 32
_CHUNK = 32


def _tc_body(xf_ref, xb_ref, w_ref, y1t_ref, y2t_ref, idx_ref, nnt_ref,
             dist_ref):
    b = pl.program_id(0)
    r = pl.program_id(1)
    xfull = xf_ref[0]
    xr = xb_ref[0]
    n = xfull.shape[1]
    w1 = w_ref[:, :64]
    w2m1 = w_ref[:, 64:] - w1

    @pl.when(r == 0)
    def _():
        xsq = xfull * xfull
        ones = jnp.ones((xfull.shape[0], 8), jnp.float32)
        nnt_ref[...] = lax.dot_general(xsq, ones, (((0,), (0,)), ((), ())),
                                       precision=lax.Precision.HIGHEST,
                                       preferred_element_type=jnp.float32)

    w1cat = jnp.concatenate([w1, w1], axis=0)
    y1t_ref[...] = lax.dot_general(xr, w1cat, (((0,), (1,)), ((), ())),
                                   preferred_element_type=jnp.float32)
    y2t_ref[...] = lax.dot_general(xr, w2m1, (((0,), (1,)), ((), ())),
                                   preferred_element_type=jnp.float32)

    rows = lax.broadcasted_iota(jnp.int32, (n, RBLK), 0)
    cols = lax.broadcasted_iota(jnp.int32, (n, RBLK), 1)
    nblk2 = n // (2 * RBLK)
    base = b * n

    def stage(dst, blk):
        xcols = xf_ref[0, :, pl.ds(pl.multiple_of(blk * RBLK, RBLK), RBLK)]
        g2 = lax.dot_general(xfull, xcols + xcols, (((0,), (0,)), ((), ())),
                             preferred_element_type=jnp.float32)
        dist_ref[dst] = jnp.where(rows == cols + blk * RBLK, NEG,
                                  g2 - nnt_ref[:, 0:1])

    @pl.when(r == 0)
    def _():
        stage(0, 0)
        stage(1, 1)

    @pl.when(r + 1 < nblk2)
    def _():
        stage((2 * r + 2) % 4, 2 * r + 2)
        stage((2 * r + 3) % 4, 2 * r + 3)

    i0 = 2 * (r % 2)
    i1 = i0 + 1
    idx_ref[0:1, :] = lax.broadcasted_iota(jnp.int32, (1, 2 * RBLK), 1) + (
        base + r * 2 * RBLK)
    jp0 = None
    jp1 = None
    for s in range(1, KNN):
        d0 = dist_ref[i0]
        d1 = dist_ref[i1]
        if jp0 is not None:
            d0 = jnp.where(rows == jp0, NEG, d0)
            dist_ref[i0] = d0
            d1 = jnp.where(rows == jp1, NEG, d1)
            dist_ref[i1] = d1
        j0 = jnp.argmax(d0, axis=0, keepdims=True).astype(jnp.int32)
        j1 = jnp.argmax(d1, axis=0, keepdims=True).astype(jnp.int32)
        idx_ref[s:s + 1, 0:RBLK] = j0 + base
        idx_ref[s:s + 1, RBLK:2 * RBLK] = j1 + base
        jp0 = j0
        jp1 = j1


def _sc_body(y1t_hbm, idx_hbm, y2t_hbm, out_hbm, idx_v, rows_v, y2_v, out_v,
             sem):
    wid = lax.axis_index("s") * 2 + lax.axis_index("c")
    pts = y1t_hbm.shape[0] // _NW
    tile_base = wid * pts
    pltpu.sync_copy(idx_hbm.at[pl.ds(0, 24), pl.ds(tile_base, pts)], idx_v)

    def chunk(ch, _):
        base = tile_base + ch * _CHUNK
        off = ch * _CHUNK
        copies = [
            pltpu.async_copy(y1t_hbm.at[idx_v.at[s, pl.ds(off, _CHUNK)]],
                             rows_v.at[s], sem)
            for s in range(KNN)
        ]
        pltpu.sync_copy(y2t_hbm.at[pl.ds(base, _CHUNK)], y2_v)
        for c in copies:
            c.wait()

        def point(p, _):
            for c in range(4):
                sl = pl.ds(c * 16, 16)
                acc = rows_v[0, p, sl]
                for s in range(1, KNN):
                    acc = jnp.maximum(acc, rows_v[s, p, sl])
                z = acc + y2_v[p, sl]
                out_v[p, sl] = jnp.where(z >= 0, z, 0.01 * z)
            return 0

        lax.fori_loop(0, _CHUNK, point, 0)
        pltpu.sync_copy(out_v, out_hbm.at[pl.ds(base, _CHUNK)])
        return 0

    lax.fori_loop(0, pts // _CHUNK, chunk, 0)


def _half(xh, W, B, b0):
    _, C, N = xh.shape
    O = W.shape[0]

    dblk = 2 * RBLK
    y1t, y2t, idxt = pl.pallas_call(
        _tc_body,
        grid=(B, N // dblk),
        in_specs=[
            pl.BlockSpec((1, C, N), lambda b, r: (b + b0, 0, 0)),
            pl.BlockSpec((1, C, dblk), lambda b, r: (b + b0, 0, r)),
            pl.BlockSpec((O, 2 * C), lambda b, r: (0, 0)),
        ],
        out_specs=[
            pl.BlockSpec((dblk, 2 * C), lambda b, r: (b * (N // dblk) + r, 0)),
            pl.BlockSpec((dblk, C), lambda b, r: (b * (N // dblk) + r, 0)),
            pl.BlockSpec((24, dblk), lambda b, r: (0, b * (N // dblk) + r)),
        ],
        out_shape=[
            jax.ShapeDtypeStruct((B * N, 2 * C), jnp.float32),
            jax.ShapeDtypeStruct((B * N, C), jnp.float32),
            jax.ShapeDtypeStruct((24, B * N), jnp.int32),
        ],
        scratch_shapes=[
            pltpu.VMEM((N, 8), jnp.float32),
            pltpu.VMEM((4, N, RBLK), jnp.float32),
        ],
        compiler_params=pltpu.CompilerParams(
            dimension_semantics=("arbitrary", "arbitrary")),
    )(xh, xh, W)

    mesh = plsc.VectorSubcoreMesh(core_axis_name="c", subcore_axis_name="s")
    outt = pl.kernel(
        _sc_body,
        mesh=mesh,
        out_type=jax.ShapeDtypeStruct((B * N, O), jnp.float32),
        scratch_types=[
            pltpu.VMEM((24, B * N // _NW), jnp.int32),
            pltpu.VMEM((KNN, _CHUNK, 2 * O), jnp.float32),
            pltpu.VMEM((_CHUNK, O), jnp.float32),
            pltpu.VMEM((_CHUNK, O), jnp.float32),
            pltpu.SemaphoreType.DMA,
        ],
    )(y1t, idxt, y2t)

    return outt.reshape(B, N, O)


def kernel(x, W):
    B, C, N = x.shape
    h = B // 4
    outs = [_half(x, W, h, i * h) for i in range(4)]
    return jnp.concatenate(outs, axis=0).transpose(0, 2, 1)


# speedup vs baseline: 1.0356x; 1.0017x over previous
import jax
import jax.numpy as jnp
from jax import lax
from jax.experimental import pallas as pl
from jax.experimental.pallas import tpu as pltpu
from jax.experimental.pallas import tpu_sc as plsc

KNN = 20
RBLK = 256
NEG = -1e30

_NW = 32
_CHUNK = 16


def _tc_body(xf_ref, xb_ref, w_ref, y1t_ref, y2t_ref, idx_ref, nnt_ref,
             dist_ref):
    b = pl.program_id(0)
    r = pl.program_id(1)
    xfull = xf_ref[0]
    xr = xb_ref[0]
    n = xfull.shape[1]
    w1 = w_ref[:, :64]
    w2m1 = w_ref[:, 64:] - w1

    @pl.when(r == 0)
    def _():
        xsq = xfull * xfull
        ones = jnp.ones((xfull.shape[0], 8), jnp.float32)
        nnt_ref[...] = lax.dot_general(xsq, ones, (((0,), (0,)), ((), ())),
                                       precision=lax.Precision.HIGHEST,
                                       preferred_element_type=jnp.float32)

    w1cat = jnp.concatenate([w1, w1], axis=0)
    y1t_ref[...] = lax.dot_general(xr, w1cat, (((0,), (1,)), ((), ())),
                                   preferred_element_type=jnp.float32)
    y2t_ref[...] = lax.dot_general(xr, w2m1, (((0,), (1,)), ((), ())),
                                   preferred_element_type=jnp.float32)

    rows = lax.broadcasted_iota(jnp.int32, (n, RBLK), 0)
    cols = lax.broadcasted_iota(jnp.int32, (n, RBLK), 1)
    nblk2 = n // (2 * RBLK)
    base = b * n

    def stage(dst, blk):
        xcols = xf_ref[0, :, pl.ds(pl.multiple_of(blk * RBLK, RBLK), RBLK)]
        g2 = lax.dot_general(xfull, xcols + xcols, (((0,), (0,)), ((), ())),
                             preferred_element_type=jnp.float32)
        dist_ref[dst] = jnp.where(rows == cols + blk * RBLK, NEG,
                                  g2 - nnt_ref[:, 0:1])

    @pl.when(r == 0)
    def _():
        stage(0, 0)
        stage(1, 1)

    @pl.when(r + 1 < nblk2)
    def _():
        stage((2 * r + 2) % 4, 2 * r + 2)
        stage((2 * r + 3) % 4, 2 * r + 3)

    i0 = 2 * (r % 2)
    i1 = i0 + 1
    idx_ref[0:1, :] = lax.broadcasted_iota(jnp.int32, (1, 2 * RBLK), 1) + (
        base + r * 2 * RBLK)
    jp0 = None
    jp1 = None
    for s in range(1, KNN):
        d0 = dist_ref[i0]
        d1 = dist_ref[i1]
        if jp0 is not None:
            d0 = jnp.where(rows == jp0, NEG, d0)
            dist_ref[i0] = d0
            d1 = jnp.where(rows == jp1, NEG, d1)
            dist_ref[i1] = d1
        j0 = jnp.argmax(d0, axis=0, keepdims=True).astype(jnp.int32)
        j1 = jnp.argmax(d1, axis=0, keepdims=True).astype(jnp.int32)
        idx_ref[s:s + 1, 0:RBLK] = j0 + base
        idx_ref[s:s + 1, RBLK:2 * RBLK] = j1 + base
        jp0 = j0
        jp1 = j1


def _sc_body(y1t_hbm, idx_hbm, y2t_hbm, out_hbm, idx_v, rows_v, y2_v, out_v,
             sem0, sem1):
    wid = lax.axis_index("s") * 2 + lax.axis_index("c")
    pts = y1t_hbm.shape[0] // _NW
    tile_base = wid * pts
    pltpu.sync_copy(idx_hbm.at[pl.ds(0, 24), pl.ds(tile_base, pts)], idx_v)
    nch = pts // _CHUNK

    def fire(ch, buf, sem):
        off = ch * _CHUNK
        for s in range(KNN):
            pltpu.async_copy(y1t_hbm.at[idx_v.at[s, pl.ds(off, _CHUNK)]],
                             rows_v.at[buf, s], sem)
        pltpu.async_copy(y2t_hbm.at[pl.ds(tile_base + off, _CHUNK)],
                         y2_v.at[buf], sem)

    def drain(buf, sem):
        for s in range(KNN):
            pltpu.make_async_copy(y1t_hbm.at[pl.ds(0, _CHUNK)],
                                  rows_v.at[buf, s], sem).wait()
        pltpu.make_async_copy(y2t_hbm.at[pl.ds(0, _CHUNK)],
                              y2_v.at[buf], sem).wait()

    def compute(buf, ch):
        base = tile_base + ch * _CHUNK

        def point(p, _):
            for c in range(4):
                sl = pl.ds(c * 16, 16)
                acc = rows_v[buf, 0, p, sl]
                for s in range(1, KNN):
                    acc = jnp.maximum(acc, rows_v[buf, s, p, sl])
                z = acc + y2_v[buf, p, sl]
                out_v[buf, p, sl] = jnp.where(z >= 0, z, 0.01 * z)
            return 0

        lax.fori_loop(0, _CHUNK, point, 0)
        pltpu.sync_copy(out_v.at[buf], out_hbm.at[pl.ds(base, _CHUNK)])

    fire(0, 0, sem0)

    def pairbody(g, _):
        ch0 = g * 2
        fire(ch0 + 1, 1, sem1)
        drain(0, sem0)
        compute(0, ch0)

        @pl.when(ch0 + 2 < nch)
        def _():
            fire(ch0 + 2, 0, sem0)

        drain(1, sem1)
        compute(1, ch0 + 1)
        return 0

    lax.fori_loop(0, nch // 2, pairbody, 0)


def _half(xh, W, B, b0):
    _, C, N = xh.shape
    O = W.shape[0]

    dblk = 2 * RBLK
    y1t, y2t, idxt = pl.pallas_call(
        _tc_body,
        grid=(B, N // dblk),
        in_specs=[
            pl.BlockSpec((1, C, N), lambda b, r: (b + b0, 0, 0)),
            pl.BlockSpec((1, C, dblk), lambda b, r: (b + b0, 0, r)),
            pl.BlockSpec((O, 2 * C), lambda b, r: (0, 0)),
        ],
        out_specs=[
            pl.BlockSpec((dblk, 2 * C), lambda b, r: (b * (N // dblk) + r, 0)),
            pl.BlockSpec((dblk, C), lambda b, r: (b * (N // dblk) + r, 0)),
            pl.BlockSpec((24, dblk), lambda b, r: (0, b * (N // dblk) + r)),
        ],
        out_shape=[
            jax.ShapeDtypeStruct((B * N, 2 * C), jnp.float32),
            jax.ShapeDtypeStruct((B * N, C), jnp.float32),
            jax.ShapeDtypeStruct((24, B * N), jnp.int32),
        ],
        scratch_shapes=[
            pltpu.VMEM((N, 8), jnp.float32),
            pltpu.VMEM((4, N, RBLK), jnp.float32),
        ],
        compiler_params=pltpu.CompilerParams(
            dimension_semantics=("arbitrary", "arbitrary")),
    )(xh, xh, W)

    mesh = plsc.VectorSubcoreMesh(core_axis_name="c", subcore_axis_name="s")
    outt = pl.kernel(
        _sc_body,
        mesh=mesh,
        out_type=jax.ShapeDtypeStruct((B * N, O), jnp.float32),
        scratch_types=[
            pltpu.VMEM((24, B * N // _NW), jnp.int32),
            pltpu.VMEM((2, KNN, _CHUNK, 2 * O), jnp.float32),
            pltpu.VMEM((2, _CHUNK, O), jnp.float32),
            pltpu.VMEM((2, _CHUNK, O), jnp.float32),
            pltpu.SemaphoreType.DMA,
            pltpu.SemaphoreType.DMA,
        ],
    )(y1t, idxt, y2t)

    return outt.reshape(B, N, O)


def kernel(x, W):
    B, C, N = x.shape
    h = B // 4
    outs = [_half(x, W, h, i * h) for i in range(4)]
    return jnp.concatenate(outs, axis=0).transpose(0, 2, 1)
